# baseline (speedup 1.0000x reference)
import functools

import jax
import jax.numpy as jnp
from jax import lax
from jax.experimental import pallas as pl
from jax.experimental.pallas import tpu as pltpu


from jax.experimental.pallas import tpu_sc as plsc


@functools.cache
def _make_sc_gather_softmax(n_neurons: int, n_sf: int, b: int):
    info = plsc.get_sparse_core_info()
    nc, ns, nl = info.num_cores, info.num_subcores, info.num_lanes
    nw = nc * ns
    b_per_w = b // nw
    nv = n_sf // nl
    mesh = plsc.VectorSubcoreMesh(core_axis_name="c", subcore_axis_name="s")

    @functools.partial(
        pl.kernel,
        mesh=mesh,
        out_type=jax.ShapeDtypeStruct((b, n_sf), jnp.float32),
        scratch_types=[
            pltpu.VMEM((b_per_w,), jnp.int32),
            pltpu.VMEM((b_per_w, n_sf), jnp.float32),
            pltpu.SemaphoreType.DMA,
        ],
    )
    def sc_kernel(table_hbm, idx_hbm, out_hbm, idx_v, rows_v, sem):
        wid = lax.axis_index("s") * nc + lax.axis_index("c")
        base = wid * b_per_w
        pltpu.sync_copy(idx_hbm.at[pl.ds(base, b_per_w)], idx_v)
        pltpu.async_copy(table_hbm.at[idx_v], rows_v, sem).wait()

        lane = lax.iota(jnp.int32, nl)
        gather_dn = lax.GatherDimensionNumbers(
            offset_dims=(), collapsed_slice_dims=(0,), start_index_map=(0,))

        def shuffle(v, sh):
            return lax.gather(v, (lane ^ sh)[:, None], gather_dn,
                              slice_sizes=(1,),
                              mode=lax.GatherScatterMode.PROMISE_IN_BOUNDS)

        def butterfly(v, op):
            sh = nl // 2
            while sh:
                v = op(v, shuffle(v, sh))
                sh //= 2
            return v

        def row_body(r, carry):
            vs = [rows_v[r, pl.ds(j * nl, nl)] for j in range(nv)]
            m = vs[0]
            for j in range(1, nv):
                m = jnp.maximum(m, vs[j])
            row_max = butterfly(m, jnp.maximum)
            es = [jnp.exp(v - row_max) for v in vs]
            acc = es[0]
            for j in range(1, nv):
                acc = acc + es[j]
            inv = 1.0 / butterfly(acc, jnp.add)
            for j in range(nv):
                rows_v[r, pl.ds(j * nl, nl)] = es[j] * inv
            return carry

        lax.fori_loop(0, b_per_w, row_body, 0)
        pltpu.sync_copy(rows_v, out_hbm.at[pl.ds(base, b_per_w)])

    return sc_kernel


def _tc_weighted_stream(xt, att):
    bsz, c_dim, s_dim, n_sf = xt.shape
    cs = c_dim * s_dim
    bb = 8
    nbuf = 8
    nchunks = bsz // bb

    def body(x_hbm, att_ref, o_ref, buf, sems):
        def start(c, slot):
            pltpu.make_async_copy(
                x_hbm.at[pl.ds(c * bb, bb)], buf.at[slot], sems.at[slot]
            ).start()

        def wait(slot):
            pltpu.make_async_copy(
                x_hbm.at[pl.ds(0, bb)], buf.at[slot], sems.at[slot]
            ).wait()

        for s in range(nbuf):
            start(s, s)

        def outer(g, carry):
            base = g * nbuf
            for s in range(nbuf):
                c = base + s
                wait(s)
                for b in range(bb):
                    row = c * bb + b
                    xb = buf[s, b].reshape(cs, n_sf)
                    w_row = att_ref[pl.ds(row, 1), :]
                    o_ref[pl.ds(row, 1), :] = lax.dot_general(
                        w_row, xb, (((1,), (1,)), ((), ())),
                        preferred_element_type=jnp.float32)
                nxt = c + nbuf

                @pl.when(nxt < nchunks)
                def _():
                    start(nxt, s)
            return carry

        lax.fori_loop(0, nchunks // nbuf, outer, 0)

    return pl.pallas_call(
        body,
        in_specs=[
            pl.BlockSpec(memory_space=pl.ANY),
            pl.BlockSpec(memory_space=pltpu.VMEM),
        ],
        out_specs=pl.BlockSpec(memory_space=pltpu.VMEM),
        out_shape=jax.ShapeDtypeStruct((bsz, cs), jnp.float32),
        scratch_shapes=[
            pltpu.VMEM((nbuf, bb, c_dim, s_dim, n_sf), jnp.float32),
            pltpu.SemaphoreType.DMA((nbuf,)),
        ],
    )(xt, att)


def kernel(x, neuron_list, alphas):
    b = x.shape[0]
    n_neurons, n_sf = alphas.shape
    xt = x.transpose(0, 2, 3, 1)
    alphas_att = _make_sc_gather_softmax(n_neurons, n_sf, b)(alphas, neuron_list)
    out = _tc_weighted_stream(xt, alphas_att)
    return out, alphas_att


# speedup vs baseline: 1.0652x; 1.0652x over previous
import functools

import jax
import jax.numpy as jnp
from jax import lax
from jax.experimental import pallas as pl
from jax.experimental.pallas import tpu as pltpu


from jax.experimental.pallas import tpu_sc as plsc


@functools.cache
def _make_sc_gather_softmax(n_neurons: int, n_sf: int, b: int):
    info = plsc.get_sparse_core_info()
    nc, ns, nl = info.num_cores, info.num_subcores, info.num_lanes
    nw = nc * ns
    b_per_w = b // nw
    nv = n_sf // nl
    mesh = plsc.VectorSubcoreMesh(core_axis_name="c", subcore_axis_name="s")

    @functools.partial(
        pl.kernel,
        mesh=mesh,
        out_type=jax.ShapeDtypeStruct((b, n_sf), jnp.float32),
        scratch_types=[
            pltpu.VMEM((b_per_w,), jnp.int32),
            pltpu.VMEM((b_per_w, n_sf), jnp.float32),
            pltpu.SemaphoreType.DMA,
        ],
    )
    def sc_kernel(table_hbm, idx_hbm, out_hbm, idx_v, rows_v, sem):
        wid = lax.axis_index("s") * nc + lax.axis_index("c")
        base = wid * b_per_w
        pltpu.sync_copy(idx_hbm.at[pl.ds(base, b_per_w)], idx_v)
        pltpu.async_copy(table_hbm.at[idx_v], rows_v, sem).wait()

        lane = lax.iota(jnp.int32, nl)
        gather_dn = lax.GatherDimensionNumbers(
            offset_dims=(), collapsed_slice_dims=(0,), start_index_map=(0,))

        def shuffle(v, sh):
            return lax.gather(v, (lane ^ sh)[:, None], gather_dn,
                              slice_sizes=(1,),
                              mode=lax.GatherScatterMode.PROMISE_IN_BOUNDS)

        def butterfly(v, op):
            sh = nl // 2
            while sh:
                v = op(v, shuffle(v, sh))
                sh //= 2
            return v

        def row_body(r, carry):
            vs = [rows_v[r, pl.ds(j * nl, nl)] for j in range(nv)]
            m = vs[0]
            for j in range(1, nv):
                m = jnp.maximum(m, vs[j])
            row_max = butterfly(m, jnp.maximum)
            es = [jnp.exp(v - row_max) for v in vs]
            acc = es[0]
            for j in range(1, nv):
                acc = acc + es[j]
            inv = 1.0 / butterfly(acc, jnp.add)
            for j in range(nv):
                rows_v[r, pl.ds(j * nl, nl)] = es[j] * inv
            return carry

        lax.fori_loop(0, b_per_w, row_body, 0)
        pltpu.sync_copy(rows_v, out_hbm.at[pl.ds(base, b_per_w)])

    return sc_kernel


def _tc_fused(xt, idx, alphas_pad):
    bsz, c_dim, s_dim, n_sf = xt.shape
    cs = c_dim * s_dim
    npad = alphas_pad.shape[0]
    bb = 8
    nbuf = 8
    nchunks = bsz // bb

    def body(x_hbm, idx_ref, a_ref, o_ref, buf, sems, att_ref):
        def start(c, slot):
            pltpu.make_async_copy(
                x_hbm.at[pl.ds(c * bb, bb)], buf.at[slot], sems.at[slot]
            ).start()

        def wait(slot):
            pltpu.make_async_copy(
                x_hbm.at[pl.ds(0, bb)], buf.at[slot], sems.at[slot]
            ).wait()

        for s in range(nbuf):
            start(s, s)

        ids_flat = idx_ref[...]
        iota_v = lax.broadcasted_iota(jnp.int32, (bsz, npad), 1)
        onehot = jnp.where(iota_v == ids_flat, 1.0, 0.0)
        gathered = jnp.dot(onehot, a_ref[...],
                           preferred_element_type=jnp.float32)
        row_max = jnp.max(gathered, axis=1, keepdims=True)
        e = jnp.exp(gathered - row_max)
        att = e / jnp.sum(e, axis=1, keepdims=True)
        att_ref[...] = att

        def outer(g, carry):
            base = g * nbuf
            for s in range(nbuf):
                c = base + s
                wait(s)
                for b in range(bb):
                    row = c * bb + b
                    xb = buf[s, b].reshape(cs, n_sf)
                    w_row = att_ref[pl.ds(row, 1), :]
                    o_ref[pl.ds(row, 1), :] = lax.dot_general(
                        w_row, xb, (((1,), (1,)), ((), ())),
                        preferred_element_type=jnp.float32)
                nxt = c + nbuf

                @pl.when(nxt < nchunks)
                def _():
                    start(nxt, s)
            return carry

        lax.fori_loop(0, nchunks // nbuf, outer, 0)

    return pl.pallas_call(
        body,
        in_specs=[
            pl.BlockSpec(memory_space=pl.ANY),
            pl.BlockSpec(memory_space=pltpu.VMEM),
            pl.BlockSpec(memory_space=pltpu.VMEM),
        ],
        out_specs=pl.BlockSpec(memory_space=pltpu.VMEM),
        out_shape=jax.ShapeDtypeStruct((bsz, cs), jnp.float32),
        scratch_shapes=[
            pltpu.VMEM((nbuf, bb, c_dim, s_dim, n_sf), jnp.float32),
            pltpu.SemaphoreType.DMA((nbuf,)),
            pltpu.VMEM((bsz, n_sf), jnp.float32),
        ],
    )(xt, idx, alphas_pad)


def kernel(x, neuron_list, alphas):
    b = x.shape[0]
    n_neurons, n_sf = alphas.shape
    npad = 64
    alphas_pad = jnp.zeros((npad, n_sf), jnp.float32).at[:n_neurons].set(alphas)
    xt = x.transpose(0, 2, 3, 1)
    idx2 = neuron_list.reshape(b, 1)
    out = _tc_fused(xt, idx2, alphas_pad)
    alphas_att = _make_sc_gather_softmax(n_neurons, n_sf, b)(alphas, neuron_list)
    return out, alphas_att
